# baseline (speedup 1.0000x reference)
import functools

import numpy as np
import jax
import jax.numpy as jnp
from jax import lax
from jax.experimental import pallas as pl
from jax.experimental.pallas import tpu as pltpu
from jax.experimental.pallas import tpu_sc as plsc

N = 10000
E = 320000
H = 128
HEADS = 4
DH = H // HEADS
NGRAPH = 64
BINS = 128

NC = 2
NS = 16
NW = NC * NS
CH = 128
NCHUNK = E // CH
GITERS = (NCHUNK + NW - 1) // NW
SITERS = (NCHUNK + NS - 1) // NS
NP = 10240
ROWS_PER_SUB = NP // NS
GCH = 64
GNCHUNK = E // GCH
GITERS2 = (GNCHUNK + NW - 1) // NW

@functools.cache
def _sc_mesh():
    return plsc.VectorSubcoreMesh(core_axis_name="c", subcore_axis_name="s",
                                  num_cores=NC, num_subcores=NS)

_DOT = functools.partial(
    lax.dot_general,
    dimension_numbers=(((1,), (0,)), ((), ())),
    preferred_element_type=jnp.float32,
    precision=lax.Precision.HIGHEST,
)


def _DOTBF(a, b):
    return lax.dot_general(a.astype(jnp.bfloat16), b.astype(jnp.bfloat16),
                           dimension_numbers=(((1,), (0,)), ((), ())),
                           preferred_element_type=jnp.float32)


def _sc_gather2_body(tbl_hbm, dst_hbm, src_hbm, od, os_,
                     idxd0, idxs0, idxd1, idxs1,
                     bd0, bs0, bd1, bs1,
                     semg0, semg1, semw0, semw1):
    wid = lax.axis_index("s") * NC + lax.axis_index("c")
    n = jnp.where(wid < GNCHUNK - (GITERS2 - 1) * NW, GITERS2, GITERS2 - 1)

    def fire(c, idxd, idxs, bd, bs, semg):
        base = c * GCH
        pltpu.sync_copy(dst_hbm.at[pl.ds(base, GCH)], idxd)
        pltpu.sync_copy(src_hbm.at[pl.ds(base, GCH)], idxs)
        pltpu.async_copy(tbl_hbm.at[idxd], bd, semg)
        pltpu.async_copy(tbl_hbm.at[idxs], bs, semg)

    def complete(c, bd, bs, semg, semw):
        pltpu.make_async_copy(tbl_hbm.at[pl.ds(0, GCH)], bd, semg).wait()
        pltpu.make_async_copy(tbl_hbm.at[pl.ds(0, GCH)], bs, semg).wait()
        base = c * GCH
        pltpu.async_copy(bd, od.at[pl.ds(base, GCH)], semw)
        pltpu.async_copy(bs, os_.at[pl.ds(base, GCH)], semw)

    fire(wid, idxd0, idxs0, bd0, bs0, semg0)

    def step(i, carry):
        c_i = wid + i * NW
        c_n = wid + (i + 1) * NW

        def do(idxd_b, idxs_b, bd_b, bs_b, semg_b, semw_b,
               bd_a, bs_a, semg_a, semw_a):
            @pl.when(i + 1 < n)
            def _():
                @pl.when(i >= 1)
                def _():
                    pltpu.make_async_copy(bd_b, od.at[pl.ds(0, GCH)],
                                          semw_b).wait()
                    pltpu.make_async_copy(bs_b, os_.at[pl.ds(0, GCH)],
                                          semw_b).wait()
                fire(c_n, idxd_b, idxs_b, bd_b, bs_b, semg_b)
            complete(c_i, bd_a, bs_a, semg_a, semw_a)

        @pl.when(i % 2 == 0)
        def _():
            do(idxd1, idxs1, bd1, bs1, semg1, semw1, bd0, bs0, semg0, semw0)

        @pl.when(i % 2 == 1)
        def _():
            do(idxd0, idxs0, bd0, bs0, semg0, semw0, bd1, bs1, semg1, semw1)

        return carry

    lax.fori_loop(0, n, step, 0)
    pltpu.make_async_copy(bd0, od.at[pl.ds(0, GCH)], semw0).wait()
    pltpu.make_async_copy(bs0, os_.at[pl.ds(0, GCH)], semw0).wait()
    pltpu.make_async_copy(bd1, od.at[pl.ds(0, GCH)], semw1).wait()
    pltpu.make_async_copy(bs1, os_.at[pl.ds(0, GCH)], semw1).wait()


def _sc_gather2(tbl, dst, src):
    out_t = tuple(jax.ShapeDtypeStruct((E, 2 * H), jnp.float32)
                  for _ in range(2))
    scratch = (
        [pltpu.VMEM((GCH,), jnp.int32)] * 4
        + [pltpu.VMEM((GCH, 2 * H), jnp.float32)] * 4
        + [pltpu.SemaphoreType.DMA] * 4
    )
    fn = pl.kernel(_sc_gather2_body, out_type=out_t, mesh=_sc_mesh(),
                   scratch_types=scratch)
    return fn(tbl, dst, src)


def _sc_scatter2_body(mc_hbm, mm_hbm, dst_hbm, z_hbm, out_hbm,
                      idx0, idx1, buf0, buf1, sem0, sem1,
                      semsc0, semsc1, shared):
    cid = lax.axis_index("c")
    sid = lax.axis_index("s")
    row0 = sid * ROWS_PER_SUB
    pltpu.sync_copy(z_hbm.at[pl.ds(row0, ROWS_PER_SUB)],
                    shared.at[pl.ds(row0, ROWS_PER_SUB)])
    plsc.subcore_barrier()
    n = jnp.where(sid < NCHUNK - (SITERS - 1) * NS, SITERS, SITERS - 1)

    def run(msg_ref):
        pltpu.async_copy(dst_hbm.at[pl.ds(sid * CH, CH)], idx0, sem0)
        pltpu.async_copy(msg_ref.at[pl.ds(sid * CH, CH)], buf0, sem0)

        def step(i, carry):
            c_next = sid + (i + 1) * NS

            def do(idx_a, buf_a, sem_a, semsc_a, idx_b, buf_b, sem_b,
                   semsc_b):
                pltpu.make_async_copy(dst_hbm.at[pl.ds(0, CH)], idx_a,
                                      sem_a).wait()
                pltpu.make_async_copy(msg_ref.at[pl.ds(0, CH)], buf_a,
                                      sem_a).wait()

                @pl.when(i + 1 < n)
                def _():
                    @pl.when(i >= 1)
                    def _():
                        pltpu.make_async_copy(
                            buf_b, shared.at[pl.ds(0, CH)], semsc_b).wait()
                    pltpu.async_copy(dst_hbm.at[pl.ds(c_next * CH, CH)],
                                     idx_b, sem_b)
                    pltpu.async_copy(msg_ref.at[pl.ds(c_next * CH, CH)],
                                     buf_b, sem_b)

                pltpu.async_copy(buf_a, shared.at[idx_a], semsc_a, add=True)

            @pl.when(i % 2 == 0)
            def _():
                do(idx0, buf0, sem0, semsc0, idx1, buf1, sem1, semsc1)

            @pl.when(i % 2 == 1)
            def _():
                do(idx1, buf1, sem1, semsc1, idx0, buf0, sem0, semsc0)

            return carry

        lax.fori_loop(0, n, step, 0)
        pltpu.make_async_copy(buf0, shared.at[pl.ds(0, CH)], semsc0).wait()
        pltpu.make_async_copy(buf1, shared.at[pl.ds(0, CH)], semsc1).wait()

    @pl.when(cid == 0)
    def _():
        run(mc_hbm)

    @pl.when(cid == 1)
    def _():
        run(mm_hbm)

    plsc.subcore_barrier()
    pltpu.sync_copy(shared.at[pl.ds(row0, ROWS_PER_SUB)],
                    out_hbm.at[pl.ds(cid * NP + row0, ROWS_PER_SUB)])


def _sc_scatter2(mc, mm, dst, zeros_nh):
    out_t = jax.ShapeDtypeStruct((2 * NP, H), jnp.float32)
    scratch = [
        pltpu.VMEM((CH,), jnp.int32),
        pltpu.VMEM((CH,), jnp.int32),
        pltpu.VMEM((CH, H), jnp.float32),
        pltpu.VMEM((CH, H), jnp.float32),
        pltpu.SemaphoreType.DMA,
        pltpu.SemaphoreType.DMA,
        pltpu.SemaphoreType.DMA,
        pltpu.SemaphoreType.DMA,
        pltpu.VMEM_SHARED((NP, H), jnp.float32),
    ]
    fn = pl.kernel(_sc_scatter2_body, out_type=out_t, mesh=_sc_mesh(),
                   scratch_types=scratch)
    return fn(mc, mm, dst, zeros_nh)


def _emb_body(oh_r, emb_r, out_r, outb_r):
    nf = _DOT(oh_r[...], emb_r[...])
    out_r[...] = nf
    outb_r[...] = jnp.concatenate([nf, nf], axis=1)


def _tc_emb(oh119, atom_emb):
    return pl.pallas_call(
        _emb_body,
        out_shape=[jax.ShapeDtypeStruct((NP, H), jnp.float32),
                   jax.ShapeDtypeStruct((NP, 2 * H), jnp.float32)],
    )(oh119, atom_emb)


BE = 2000


def _ef_body(ea_r, cen_r, gam_r, w1_r, b1_r, w2_r, b2_r, out_r):
    ea = ea_r[...]
    d = jnp.sqrt(jnp.sum(ea * ea, axis=1, keepdims=True))
    t = gam_r[...] * (d - cen_r[...])
    rbf = jnp.exp(-(t * t))
    h1 = _DOTBF(rbf, w1_r[...]) + b1_r[...]
    sp = jnp.maximum(h1, 0.0) + jnp.log1p(jnp.exp(-jnp.abs(h1)))
    out_r[...] = _DOTBF(sp, w2_r[...]) + b2_r[...]


def _tc_ef(edge_attr, w1, b1, w2, b2):
    nb = E // BE
    centers = jnp.linspace(0.0, 8.0, BINS).reshape(1, BINS)
    gamma = (1.0 / (centers[0, 1] - centers[0, 0])).reshape(1, 1)
    return pl.pallas_call(
        _ef_body,
        grid=(nb,),
        in_specs=[
            pl.BlockSpec((BE, 3), lambda i: (i, 0)),
            pl.BlockSpec((1, BINS), lambda i: (0, 0)),
            pl.BlockSpec((1, 1), lambda i: (0, 0)),
            pl.BlockSpec((BINS, H), lambda i: (0, 0)),
            pl.BlockSpec((1, H), lambda i: (0, 0)),
            pl.BlockSpec((H, H), lambda i: (0, 0)),
            pl.BlockSpec((1, H), lambda i: (0, 0)),
        ],
        out_specs=pl.BlockSpec((BE, H), lambda i: (i, 0)),
        out_shape=jax.ShapeDtypeStruct((E, H), jnp.float32),
    )(edge_attr, centers, gamma, w1, b1.reshape(1, H), w2, b2.reshape(1, H))


def _edge_body(gd_r, gs_r, ef_r,
               wg_r, bg_r, wm_r, bm_r, wq_r, bq_r, wk_r, bk_r,
               wv_r, bv_r, we_r, be_r, mc_r, mm_r):
    gd = gd_r[...]
    gs = gs_r[...]
    xd = gd[:, 0:H]
    xs = gs[:, 0:H]
    e = ef_r[...]
    h = jnp.concatenate([xd, xs, e], axis=1)
    g = _DOTBF(h, wg_r[...]) + bg_r[...]
    m = _DOTBF(h, wm_r[...]) + bm_r[...]
    gate = jax.nn.sigmoid(g)
    msg = m * jax.nn.sigmoid(m)
    mc_r[...] = gate * msg

    md = gd[:, H:2 * H]
    ms = gs[:, H:2 * H]
    q = _DOTBF(md, wq_r[...]) + bq_r[...]
    k = _DOTBF(ms, wk_r[...]) + bk_r[...]
    v = _DOTBF(ms, wv_r[...]) + bv_r[...]
    ee = _DOTBF(e, we_r[...]) + be_r[...]
    t = q * (k + ee)
    lane = lax.broadcasted_iota(jnp.int32, (H, HEADS), 0)
    head = lax.broadcasted_iota(jnp.int32, (H, HEADS), 1)
    g_mat = (lane // DH == head).astype(jnp.float32)
    lane_t = lax.broadcasted_iota(jnp.int32, (HEADS, H), 1)
    head_t = lax.broadcasted_iota(jnp.int32, (HEADS, H), 0)
    g_mat_t = (lane_t // DH == head_t).astype(jnp.float32)
    s = _DOT(t, g_mat) * (1.0 / np.sqrt(DH))
    a = jax.nn.sigmoid(s)
    a128 = _DOT(a, g_mat_t)
    mm_r[...] = a128 * (v + ee)


def _tc_edge(gd, gs, ef, lp):
    nb = E // BE
    full = lambda shape: pl.BlockSpec(shape, lambda i: (0, 0))
    blk = pl.BlockSpec((BE, H), lambda i: (i, 0))
    blk2 = pl.BlockSpec((BE, 2 * H), lambda i: (i, 0))
    return pl.pallas_call(
        _edge_body,
        grid=(nb,),
        in_specs=[blk2, blk2, blk,
                  full((3 * H, H)), full((1, H)),
                  full((3 * H, H)), full((1, H)),
                  full((H, H)), full((1, H)),
                  full((H, H)), full((1, H)),
                  full((H, H)), full((1, H)),
                  full((H, H)), full((1, H))],
        out_specs=[blk, blk],
        out_shape=[jax.ShapeDtypeStruct((E, H), jnp.float32),
                   jax.ShapeDtypeStruct((E, H), jnp.float32)],
    )(gd, gs, ef,
      lp['cart_gate_w'], lp['cart_gate_b'].reshape(1, H),
      lp['cart_msg_w'], lp['cart_msg_b'].reshape(1, H),
      lp['mat_q_w'], lp['mat_q_b'].reshape(1, H),
      lp['mat_k_w'], lp['mat_k_b'].reshape(1, H),
      lp['mat_v_w'], lp['mat_v_b'].reshape(1, H),
      lp['mat_e_w'], lp['mat_e_b'].reshape(1, H))


BN = 2048


def _nodeA_body(xc_r, xm_r, aggc_r, aggm_r, oh_r, wbeta_r, bbeta_r,
                xc0_r, xm0_r, sc_r, sm_r):
    i = pl.program_id(0)
    xc0 = xc_r[...] + aggc_r[0]
    aggm = aggm_r[0]
    xm_in = xm_r[...]
    hb = jnp.concatenate([aggm, xm_in, aggm - xm_in], axis=1)
    beta = jax.nn.sigmoid(_DOTBF(hb, wbeta_r[...]) + bbeta_r[...])
    xm0 = beta * xm_in + (1.0 - beta) * aggm
    xc0_r[...] = xc0
    xm0_r[...] = xm0

    @pl.when(i == 0)
    def _():
        sc_r[...] = jnp.zeros_like(sc_r)
        sm_r[...] = jnp.zeros_like(sm_r)

    oh = oh_r[...]
    sc_r[...] += _DOT(oh, xc0)
    sm_r[...] += _DOT(oh, xm0)


def _nodeB_body(xc0_r, xm0_r, oh_r, ohT_r, sc_r, sm_r, ic_r,
                msc_r, msm_r, oc_r, om_r, vc_r, vm_r):
    i = pl.program_id(0)
    ic = ic_r[...]
    mean_c = sc_r[...] * ic
    mean_m = sm_r[...] * ic
    ohT = ohT_r[...]
    out_c = xc0_r[...] - msc_r[...] * _DOT(ohT, mean_c)
    out_m = xm0_r[...] - msm_r[...] * _DOT(ohT, mean_m)
    oc_r[...] = out_c
    om_r[...] = out_m

    @pl.when(i == 0)
    def _():
        vc_r[...] = jnp.zeros_like(vc_r)
        vm_r[...] = jnp.zeros_like(vm_r)

    oh = oh_r[...]
    vc_r[...] += _DOT(oh, out_c * out_c)
    vm_r[...] += _DOT(oh, out_m * out_m)


def _nodeC_body(oc_r, om_r, ohT_r, vc_r, vm_r, ic_r,
                wc_r, bc_r, wm_r, bm_r, wmix_r, bmix_r,
                xcart_r, xmat_r, xcmb_r):
    ic = ic_r[...]
    ohT = ohT_r[...]
    var_c = vc_r[...] * ic
    var_m = vm_r[...] * ic
    xc = (wc_r[...] * oc_r[...] / jnp.sqrt(_DOT(ohT, var_c) + 1e-5)
          + bc_r[...])
    xm = (wm_r[...] * om_r[...] / jnp.sqrt(_DOT(ohT, var_m) + 1e-5)
          + bm_r[...])
    hx = jnp.concatenate([xc, xm], axis=1)
    gate = jax.nn.sigmoid(_DOTBF(hx, wmix_r[...]) + bmix_r[...])
    xo = gate * xc + (1.0 - gate) * xm
    nxc = xc + 0.5 * xo
    nxm = xm + 0.5 * xo
    xcart_r[...] = nxc
    xmat_r[...] = nxm
    xcmb_r[...] = jnp.concatenate([nxc, nxm], axis=1)


def _tc_node(x_cart, x_mat, agg, oh, ohT, inv_cnt, lp):
    nb = NP // BN
    blk = pl.BlockSpec((BN, H), lambda i: (i, 0))
    oh_blk = pl.BlockSpec((NGRAPH, BN), lambda i: (0, i))
    ohT_blk = pl.BlockSpec((BN, NGRAPH), lambda i: (i, 0))
    acc = pl.BlockSpec((NGRAPH, H), lambda i: (0, 0))
    full = lambda shape: pl.BlockSpec(shape, lambda i: (0, 0))
    agg3 = agg.reshape(2, NP, H)

    xc0, xm0, s_c, s_m = pl.pallas_call(
        _nodeA_body,
        grid=(nb,),
        in_specs=[blk, blk,
                  pl.BlockSpec((1, BN, H), lambda i: (0, i, 0)),
                  pl.BlockSpec((1, BN, H), lambda i: (1, i, 0)),
                  oh_blk, full((3 * H, 1)), full((1, 1))],
        out_specs=[blk, blk, acc, acc],
        out_shape=[jax.ShapeDtypeStruct((NP, H), jnp.float32),
                   jax.ShapeDtypeStruct((NP, H), jnp.float32),
                   jax.ShapeDtypeStruct((NGRAPH, H), jnp.float32),
                   jax.ShapeDtypeStruct((NGRAPH, H), jnp.float32)],
    )(x_cart, x_mat, agg3, agg3, oh,
      lp['mat_beta_w'], lp['mat_beta_b'].reshape(1, 1))

    out_c, out_m, v_c, v_m = pl.pallas_call(
        _nodeB_body,
        grid=(nb,),
        in_specs=[blk, blk, oh_blk, ohT_blk,
                  full((NGRAPH, H)), full((NGRAPH, H)), full((NGRAPH, 1)),
                  full((1, H)), full((1, H))],
        out_specs=[blk, blk, acc, acc],
        out_shape=[jax.ShapeDtypeStruct((NP, H), jnp.float32),
                   jax.ShapeDtypeStruct((NP, H), jnp.float32),
                   jax.ShapeDtypeStruct((NGRAPH, H), jnp.float32),
                   jax.ShapeDtypeStruct((NGRAPH, H), jnp.float32)],
    )(xc0, xm0, oh, ohT, s_c, s_m, inv_cnt,
      lp['gn_cart_ms'].reshape(1, H), lp['gn_mat_ms'].reshape(1, H))

    return pl.pallas_call(
        _nodeC_body,
        grid=(nb,),
        in_specs=[blk, blk, ohT_blk,
                  full((NGRAPH, H)), full((NGRAPH, H)), full((NGRAPH, 1)),
                  full((1, H)), full((1, H)), full((1, H)), full((1, H)),
                  full((2 * H, H)), full((1, H))],
        out_specs=[blk, blk, pl.BlockSpec((BN, 2 * H), lambda i: (i, 0))],
        out_shape=[jax.ShapeDtypeStruct((NP, H), jnp.float32),
                   jax.ShapeDtypeStruct((NP, H), jnp.float32),
                   jax.ShapeDtypeStruct((NP, 2 * H), jnp.float32)],
    )(out_c, out_m, ohT, v_c, v_m, inv_cnt,
      lp['gn_cart_w'].reshape(1, H), lp['gn_cart_b'].reshape(1, H),
      lp['gn_mat_w'].reshape(1, H), lp['gn_mat_b'].reshape(1, H),
      lp['mix_w'], lp['mix_b'].reshape(1, H))


def _readout_body(xc_r, xm_r, oh_r, ic_r, w1_r, b1_r, w2_r, b2_r, out_r):
    xf = (xc_r[...] + xm_r[...]) * 0.5
    feats = _DOT(oh_r[...], xf) * ic_r[...]
    h1 = _DOTBF(feats, w1_r[...]) + b1_r[...]
    h1 = h1 * jax.nn.sigmoid(h1)
    out_r[...] = _DOTBF(h1, w2_r[...]) + b2_r[...]


def _tc_readout(x_cart, x_mat, oh, inv_cnt, w1, b1, w2, b2):
    return pl.pallas_call(
        _readout_body,
        out_shape=jax.ShapeDtypeStruct((NGRAPH, 1), jnp.float32),
    )(x_cart, x_mat, oh, inv_cnt, w1, b1.reshape(1, H), w2, b2.reshape(1, 1))


def kernel(x_atom, edge_index, edge_attr, batch, params):
    src = edge_index[0].astype(jnp.int32)
    dst = edge_index[1].astype(jnp.int32)
    batch_i = jnp.pad(batch.astype(jnp.int32), (0, NP - N),
                      constant_values=NGRAPH)

    oh = (batch_i[None, :] == jnp.arange(NGRAPH, dtype=jnp.int32)[:, None]
          ).astype(jnp.float32)
    ohT = oh.T
    cnt = jnp.sum(oh, axis=1, keepdims=True)
    inv_cnt = 1.0 / jnp.maximum(cnt, 1.0)

    atom_p = jnp.pad(x_atom.astype(jnp.int32), (0, NP - N),
                     constant_values=-1)
    oh119 = (atom_p[:, None]
             == jnp.arange(119, dtype=jnp.int32)[None, :]).astype(jnp.float32)
    node_f, node_cmb = _tc_emb(oh119, params['atom_emb'])

    ef = _tc_ef(edge_attr, params['rbf_w1'], params['rbf_b1'],
                params['rbf_w2'], params['rbf_b2'])

    zeros_nh = jnp.zeros((NP, H), jnp.float32)

    x_cart = node_f
    x_mat = node_f
    x_cmb = node_cmb
    for lp in params['layers']:
        gd, gs = _sc_gather2(x_cmb, dst, src)
        mc, mm = _tc_edge(gd, gs, ef, lp)
        agg = _sc_scatter2(mc, mm, dst, zeros_nh)
        x_cart, x_mat, x_cmb = _tc_node(
            x_cart, x_mat, agg, oh, ohT, inv_cnt, lp)

    return _tc_readout(x_cart, x_mat, oh, inv_cnt,
                       params['fc1_w'], params['fc1_b'],
                       params['fc2_w'], params['fc2_b'])


# speedup vs baseline: 1.0918x; 1.0918x over previous
import functools

import numpy as np
import jax
import jax.numpy as jnp
from jax import lax
from jax.experimental import pallas as pl
from jax.experimental.pallas import tpu as pltpu
from jax.experimental.pallas import tpu_sc as plsc

N = 10000
E = 320000
H = 128
HEADS = 4
DH = H // HEADS
NGRAPH = 64
BINS = 128

NC = 2
NS = 16
NW = NC * NS
CH = 128
NCHUNK = E // CH
GITERS = (NCHUNK + NW - 1) // NW
SITERS = (NCHUNK + NS - 1) // NS
NP = 10240
ROWS_PER_SUB = NP // NS
GCH = 128
GNCHUNK = E // GCH
GITERS2 = (GNCHUNK + NW - 1) // NW

@functools.cache
def _sc_mesh():
    return plsc.VectorSubcoreMesh(core_axis_name="c", subcore_axis_name="s",
                                  num_cores=NC, num_subcores=NS)

_DOT = functools.partial(
    lax.dot_general,
    dimension_numbers=(((1,), (0,)), ((), ())),
    preferred_element_type=jnp.float32,
    precision=lax.Precision.HIGHEST,
)


def _DOTBF(a, b):
    return lax.dot_general(a.astype(jnp.bfloat16), b.astype(jnp.bfloat16),
                           dimension_numbers=(((1,), (0,)), ((), ())),
                           preferred_element_type=jnp.float32)


def _sc_gather2_body(tbl_hbm, dst_hbm, src_hbm, od, os_,
                     idxd0, idxs0, idxd1, idxs1,
                     bd0, bs0, bd1, bs1,
                     semg0, semg1, semw0, semw1):
    wid = lax.axis_index("s") * NC + lax.axis_index("c")
    n = jnp.where(wid < GNCHUNK - (GITERS2 - 1) * NW, GITERS2, GITERS2 - 1)

    def fire(c, idxd, idxs, bd, bs, semg):
        base = c * GCH
        pltpu.sync_copy(dst_hbm.at[pl.ds(base, GCH)], idxd)
        pltpu.sync_copy(src_hbm.at[pl.ds(base, GCH)], idxs)
        pltpu.async_copy(tbl_hbm.at[idxd], bd, semg)
        pltpu.async_copy(tbl_hbm.at[idxs], bs, semg)

    def complete(c, bd, bs, semg, semw):
        pltpu.make_async_copy(tbl_hbm.at[pl.ds(0, GCH)], bd, semg).wait()
        pltpu.make_async_copy(tbl_hbm.at[pl.ds(0, GCH)], bs, semg).wait()
        base = c * GCH
        pltpu.async_copy(bd, od.at[pl.ds(base, GCH)], semw)
        pltpu.async_copy(bs, os_.at[pl.ds(base, GCH)], semw)

    fire(wid, idxd0, idxs0, bd0, bs0, semg0)

    def step(i, carry):
        c_i = wid + i * NW
        c_n = wid + (i + 1) * NW

        def do(idxd_b, idxs_b, bd_b, bs_b, semg_b, semw_b,
               bd_a, bs_a, semg_a, semw_a):
            @pl.when(i + 1 < n)
            def _():
                @pl.when(i >= 1)
                def _():
                    pltpu.make_async_copy(bd_b, od.at[pl.ds(0, GCH)],
                                          semw_b).wait()
                    pltpu.make_async_copy(bs_b, os_.at[pl.ds(0, GCH)],
                                          semw_b).wait()
                fire(c_n, idxd_b, idxs_b, bd_b, bs_b, semg_b)
            complete(c_i, bd_a, bs_a, semg_a, semw_a)

        @pl.when(i % 2 == 0)
        def _():
            do(idxd1, idxs1, bd1, bs1, semg1, semw1, bd0, bs0, semg0, semw0)

        @pl.when(i % 2 == 1)
        def _():
            do(idxd0, idxs0, bd0, bs0, semg0, semw0, bd1, bs1, semg1, semw1)

        return carry

    lax.fori_loop(0, n, step, 0)
    pltpu.make_async_copy(bd0, od.at[pl.ds(0, GCH)], semw0).wait()
    pltpu.make_async_copy(bs0, os_.at[pl.ds(0, GCH)], semw0).wait()
    pltpu.make_async_copy(bd1, od.at[pl.ds(0, GCH)], semw1).wait()
    pltpu.make_async_copy(bs1, os_.at[pl.ds(0, GCH)], semw1).wait()


def _sc_gather2(tbl, dst, src):
    out_t = tuple(jax.ShapeDtypeStruct((E, H), jnp.int32)
                  for _ in range(2))
    scratch = (
        [pltpu.VMEM((GCH,), jnp.int32)] * 4
        + [pltpu.VMEM((GCH, H), jnp.int32)] * 4
        + [pltpu.SemaphoreType.DMA] * 4
    )
    fn = pl.kernel(_sc_gather2_body, out_type=out_t, mesh=_sc_mesh(),
                   scratch_types=scratch)
    return fn(tbl, dst, src)


def _sc_scatter2_body(mc_hbm, mm_hbm, dst_hbm, z_hbm, out_hbm,
                      idx0, idx1, buf0, buf1, sem0, sem1,
                      semsc0, semsc1, shared):
    cid = lax.axis_index("c")
    sid = lax.axis_index("s")
    row0 = sid * ROWS_PER_SUB
    pltpu.sync_copy(z_hbm.at[pl.ds(row0, ROWS_PER_SUB)],
                    shared.at[pl.ds(row0, ROWS_PER_SUB)])
    plsc.subcore_barrier()
    n = jnp.where(sid < NCHUNK - (SITERS - 1) * NS, SITERS, SITERS - 1)

    def run(msg_ref):
        pltpu.async_copy(dst_hbm.at[pl.ds(sid * CH, CH)], idx0, sem0)
        pltpu.async_copy(msg_ref.at[pl.ds(sid * CH, CH)], buf0, sem0)

        def step(i, carry):
            c_next = sid + (i + 1) * NS

            def do(idx_a, buf_a, sem_a, semsc_a, idx_b, buf_b, sem_b,
                   semsc_b):
                pltpu.make_async_copy(dst_hbm.at[pl.ds(0, CH)], idx_a,
                                      sem_a).wait()
                pltpu.make_async_copy(msg_ref.at[pl.ds(0, CH)], buf_a,
                                      sem_a).wait()

                @pl.when(i + 1 < n)
                def _():
                    @pl.when(i >= 1)
                    def _():
                        pltpu.make_async_copy(
                            buf_b, shared.at[pl.ds(0, CH)], semsc_b).wait()
                    pltpu.async_copy(dst_hbm.at[pl.ds(c_next * CH, CH)],
                                     idx_b, sem_b)
                    pltpu.async_copy(msg_ref.at[pl.ds(c_next * CH, CH)],
                                     buf_b, sem_b)

                pltpu.async_copy(buf_a, shared.at[idx_a], semsc_a, add=True)

            @pl.when(i % 2 == 0)
            def _():
                do(idx0, buf0, sem0, semsc0, idx1, buf1, sem1, semsc1)

            @pl.when(i % 2 == 1)
            def _():
                do(idx1, buf1, sem1, semsc1, idx0, buf0, sem0, semsc0)

            return carry

        lax.fori_loop(0, n, step, 0)
        pltpu.make_async_copy(buf0, shared.at[pl.ds(0, CH)], semsc0).wait()
        pltpu.make_async_copy(buf1, shared.at[pl.ds(0, CH)], semsc1).wait()

    @pl.when(cid == 0)
    def _():
        run(mc_hbm)

    @pl.when(cid == 1)
    def _():
        run(mm_hbm)

    plsc.subcore_barrier()
    pltpu.sync_copy(shared.at[pl.ds(row0, ROWS_PER_SUB)],
                    out_hbm.at[pl.ds(cid * NP + row0, ROWS_PER_SUB)])


def _sc_scatter2(mc, mm, dst, zeros_nh):
    out_t = jax.ShapeDtypeStruct((2 * NP, H), jnp.float32)
    scratch = [
        pltpu.VMEM((CH,), jnp.int32),
        pltpu.VMEM((CH,), jnp.int32),
        pltpu.VMEM((CH, H), jnp.float32),
        pltpu.VMEM((CH, H), jnp.float32),
        pltpu.SemaphoreType.DMA,
        pltpu.SemaphoreType.DMA,
        pltpu.SemaphoreType.DMA,
        pltpu.SemaphoreType.DMA,
        pltpu.VMEM_SHARED((NP, H), jnp.float32),
    ]
    fn = pl.kernel(_sc_scatter2_body, out_type=out_t, mesh=_sc_mesh(),
                   scratch_types=scratch)
    return fn(mc, mm, dst, zeros_nh)


def _emb_body(oh_r, emb_r, out_r, outb_r):
    nf = _DOT(oh_r[...], emb_r[...])
    out_r[...] = nf
    outb_r[...] = jnp.concatenate([nf, nf], axis=1).astype(jnp.bfloat16)


def _tc_emb(oh119, atom_emb):
    return pl.pallas_call(
        _emb_body,
        out_shape=[jax.ShapeDtypeStruct((NP, H), jnp.float32),
                   jax.ShapeDtypeStruct((NP, 2 * H), jnp.bfloat16)],
    )(oh119, atom_emb)


BE = 2000


def _ef_body(ea_r, cen_r, gam_r, w1_r, b1_r, w2_r, b2_r, out_r):
    ea = ea_r[...]
    d = jnp.sqrt(jnp.sum(ea * ea, axis=1, keepdims=True))
    t = gam_r[...] * (d - cen_r[...])
    rbf = jnp.exp(-(t * t))
    h1 = _DOTBF(rbf, w1_r[...]) + b1_r[...]
    sp = jnp.maximum(h1, 0.0) + jnp.log1p(jnp.exp(-jnp.abs(h1)))
    out_r[...] = _DOTBF(sp, w2_r[...]) + b2_r[...]


def _tc_ef(edge_attr, w1, b1, w2, b2):
    nb = E // BE
    centers = jnp.linspace(0.0, 8.0, BINS).reshape(1, BINS)
    gamma = (1.0 / (centers[0, 1] - centers[0, 0])).reshape(1, 1)
    return pl.pallas_call(
        _ef_body,
        grid=(nb,),
        in_specs=[
            pl.BlockSpec((BE, 3), lambda i: (i, 0)),
            pl.BlockSpec((1, BINS), lambda i: (0, 0)),
            pl.BlockSpec((1, 1), lambda i: (0, 0)),
            pl.BlockSpec((BINS, H), lambda i: (0, 0)),
            pl.BlockSpec((1, H), lambda i: (0, 0)),
            pl.BlockSpec((H, H), lambda i: (0, 0)),
            pl.BlockSpec((1, H), lambda i: (0, 0)),
        ],
        out_specs=pl.BlockSpec((BE, H), lambda i: (i, 0)),
        out_shape=jax.ShapeDtypeStruct((E, H), jnp.float32),
    )(edge_attr, centers, gamma, w1, b1.reshape(1, H), w2, b2.reshape(1, H))


def _edge_body(gd_r, gs_r, ef_r,
               wg_r, bg_r, wm_r, bm_r, wq_r, bq_r, wk_r, bk_r,
               wv_r, bv_r, we_r, be_r, mc_r, mm_r):
    wu_d = lax.bitcast_convert_type(gd_r[...], jnp.uint32)
    wu_s = lax.bitcast_convert_type(gs_r[...], jnp.uint32)
    lo = lambda u: lax.bitcast_convert_type(u << 16, jnp.float32)
    hi = lambda u: lax.bitcast_convert_type(u & jnp.uint32(0xFFFF0000),
                                            jnp.float32)
    half = H // 2
    xd = jnp.concatenate([lo(wu_d[:, 0:half]), hi(wu_d[:, 0:half])], axis=1)
    xs = jnp.concatenate([lo(wu_s[:, 0:half]), hi(wu_s[:, 0:half])], axis=1)
    e = ef_r[...]
    h = jnp.concatenate([xd, xs, e], axis=1)
    g = _DOTBF(h, wg_r[...]) + bg_r[...]
    m = _DOTBF(h, wm_r[...]) + bm_r[...]
    gate = jax.nn.sigmoid(g)
    msg = m * jax.nn.sigmoid(m)
    mc_r[...] = gate * msg

    md = jnp.concatenate([lo(wu_d[:, half:H]), hi(wu_d[:, half:H])], axis=1)
    ms = jnp.concatenate([lo(wu_s[:, half:H]), hi(wu_s[:, half:H])], axis=1)
    q = _DOTBF(md, wq_r[...]) + bq_r[...]
    k = _DOTBF(ms, wk_r[...]) + bk_r[...]
    v = _DOTBF(ms, wv_r[...]) + bv_r[...]
    ee = _DOTBF(e, we_r[...]) + be_r[...]
    t = q * (k + ee)
    lane = lax.broadcasted_iota(jnp.int32, (H, HEADS), 0)
    head = lax.broadcasted_iota(jnp.int32, (H, HEADS), 1)
    g_mat = (lane // DH == head).astype(jnp.float32)
    lane_t = lax.broadcasted_iota(jnp.int32, (HEADS, H), 1)
    head_t = lax.broadcasted_iota(jnp.int32, (HEADS, H), 0)
    g_mat_t = (lane_t // DH == head_t).astype(jnp.float32)
    s = _DOT(t, g_mat) * (1.0 / np.sqrt(DH))
    a = jax.nn.sigmoid(s)
    a128 = _DOT(a, g_mat_t)
    mm_r[...] = a128 * (v + ee)


def _tc_edge(gd, gs, ef, lp):
    nb = E // BE
    full = lambda shape: pl.BlockSpec(shape, lambda i: (0, 0))
    blk = pl.BlockSpec((BE, H), lambda i: (i, 0))
    blk2 = pl.BlockSpec((BE, H), lambda i: (i, 0))
    perm = np.concatenate([np.arange(0, H, 2), np.arange(1, H, 2)])
    permg = np.concatenate([perm, H + perm, np.arange(2 * H, 3 * H)])
    return pl.pallas_call(
        _edge_body,
        grid=(nb,),
        in_specs=[blk2, blk2, blk,
                  full((3 * H, H)), full((1, H)),
                  full((3 * H, H)), full((1, H)),
                  full((H, H)), full((1, H)),
                  full((H, H)), full((1, H)),
                  full((H, H)), full((1, H)),
                  full((H, H)), full((1, H))],
        out_specs=[blk, blk],
        out_shape=[jax.ShapeDtypeStruct((E, H), jnp.float32),
                   jax.ShapeDtypeStruct((E, H), jnp.float32)],
    )(gd, gs, ef,
      lp['cart_gate_w'][permg], lp['cart_gate_b'].reshape(1, H),
      lp['cart_msg_w'][permg], lp['cart_msg_b'].reshape(1, H),
      lp['mat_q_w'][perm], lp['mat_q_b'].reshape(1, H),
      lp['mat_k_w'][perm], lp['mat_k_b'].reshape(1, H),
      lp['mat_v_w'][perm], lp['mat_v_b'].reshape(1, H),
      lp['mat_e_w'], lp['mat_e_b'].reshape(1, H))


BN = 2048


def _nodeA_body(xc_r, xm_r, aggc_r, aggm_r, oh_r, wbeta_r, bbeta_r,
                xc0_r, xm0_r, sc_r, sm_r):
    i = pl.program_id(0)
    xc0 = xc_r[...] + aggc_r[0]
    aggm = aggm_r[0]
    xm_in = xm_r[...]
    hb = jnp.concatenate([aggm, xm_in, aggm - xm_in], axis=1)
    beta = jax.nn.sigmoid(_DOTBF(hb, wbeta_r[...]) + bbeta_r[...])
    xm0 = beta * xm_in + (1.0 - beta) * aggm
    xc0_r[...] = xc0
    xm0_r[...] = xm0

    @pl.when(i == 0)
    def _():
        sc_r[...] = jnp.zeros_like(sc_r)
        sm_r[...] = jnp.zeros_like(sm_r)

    oh = oh_r[...]
    sc_r[...] += _DOT(oh, xc0)
    sm_r[...] += _DOT(oh, xm0)


def _nodeB_body(xc0_r, xm0_r, oh_r, ohT_r, sc_r, sm_r, ic_r,
                msc_r, msm_r, oc_r, om_r, vc_r, vm_r):
    i = pl.program_id(0)
    ic = ic_r[...]
    mean_c = sc_r[...] * ic
    mean_m = sm_r[...] * ic
    ohT = ohT_r[...]
    out_c = xc0_r[...] - msc_r[...] * _DOT(ohT, mean_c)
    out_m = xm0_r[...] - msm_r[...] * _DOT(ohT, mean_m)
    oc_r[...] = out_c
    om_r[...] = out_m

    @pl.when(i == 0)
    def _():
        vc_r[...] = jnp.zeros_like(vc_r)
        vm_r[...] = jnp.zeros_like(vm_r)

    oh = oh_r[...]
    vc_r[...] += _DOT(oh, out_c * out_c)
    vm_r[...] += _DOT(oh, out_m * out_m)


def _nodeC_body(oc_r, om_r, ohT_r, vc_r, vm_r, ic_r,
                wc_r, bc_r, wm_r, bm_r, wmix_r, bmix_r,
                xcart_r, xmat_r, xcmb_r):
    ic = ic_r[...]
    ohT = ohT_r[...]
    var_c = vc_r[...] * ic
    var_m = vm_r[...] * ic
    xc = (wc_r[...] * oc_r[...] / jnp.sqrt(_DOT(ohT, var_c) + 1e-5)
          + bc_r[...])
    xm = (wm_r[...] * om_r[...] / jnp.sqrt(_DOT(ohT, var_m) + 1e-5)
          + bm_r[...])
    hx = jnp.concatenate([xc, xm], axis=1)
    gate = jax.nn.sigmoid(_DOTBF(hx, wmix_r[...]) + bmix_r[...])
    xo = gate * xc + (1.0 - gate) * xm
    nxc = xc + 0.5 * xo
    nxm = xm + 0.5 * xo
    xcart_r[...] = nxc
    xmat_r[...] = nxm
    xcmb_r[...] = jnp.concatenate([nxc, nxm], axis=1).astype(jnp.bfloat16)


def _tc_node(x_cart, x_mat, agg, oh, ohT, inv_cnt, lp):
    nb = NP // BN
    blk = pl.BlockSpec((BN, H), lambda i: (i, 0))
    oh_blk = pl.BlockSpec((NGRAPH, BN), lambda i: (0, i))
    ohT_blk = pl.BlockSpec((BN, NGRAPH), lambda i: (i, 0))
    acc = pl.BlockSpec((NGRAPH, H), lambda i: (0, 0))
    full = lambda shape: pl.BlockSpec(shape, lambda i: (0, 0))
    agg3 = agg.reshape(2, NP, H)

    xc0, xm0, s_c, s_m = pl.pallas_call(
        _nodeA_body,
        grid=(nb,),
        in_specs=[blk, blk,
                  pl.BlockSpec((1, BN, H), lambda i: (0, i, 0)),
                  pl.BlockSpec((1, BN, H), lambda i: (1, i, 0)),
                  oh_blk, full((3 * H, 1)), full((1, 1))],
        out_specs=[blk, blk, acc, acc],
        out_shape=[jax.ShapeDtypeStruct((NP, H), jnp.float32),
                   jax.ShapeDtypeStruct((NP, H), jnp.float32),
                   jax.ShapeDtypeStruct((NGRAPH, H), jnp.float32),
                   jax.ShapeDtypeStruct((NGRAPH, H), jnp.float32)],
    )(x_cart, x_mat, agg3, agg3, oh,
      lp['mat_beta_w'], lp['mat_beta_b'].reshape(1, 1))

    out_c, out_m, v_c, v_m = pl.pallas_call(
        _nodeB_body,
        grid=(nb,),
        in_specs=[blk, blk, oh_blk, ohT_blk,
                  full((NGRAPH, H)), full((NGRAPH, H)), full((NGRAPH, 1)),
                  full((1, H)), full((1, H))],
        out_specs=[blk, blk, acc, acc],
        out_shape=[jax.ShapeDtypeStruct((NP, H), jnp.float32),
                   jax.ShapeDtypeStruct((NP, H), jnp.float32),
                   jax.ShapeDtypeStruct((NGRAPH, H), jnp.float32),
                   jax.ShapeDtypeStruct((NGRAPH, H), jnp.float32)],
    )(xc0, xm0, oh, ohT, s_c, s_m, inv_cnt,
      lp['gn_cart_ms'].reshape(1, H), lp['gn_mat_ms'].reshape(1, H))

    return pl.pallas_call(
        _nodeC_body,
        grid=(nb,),
        in_specs=[blk, blk, ohT_blk,
                  full((NGRAPH, H)), full((NGRAPH, H)), full((NGRAPH, 1)),
                  full((1, H)), full((1, H)), full((1, H)), full((1, H)),
                  full((2 * H, H)), full((1, H))],
        out_specs=[blk, blk, pl.BlockSpec((BN, 2 * H), lambda i: (i, 0))],
        out_shape=[jax.ShapeDtypeStruct((NP, H), jnp.float32),
                   jax.ShapeDtypeStruct((NP, H), jnp.float32),
                   jax.ShapeDtypeStruct((NP, 2 * H), jnp.bfloat16)],
    )(out_c, out_m, ohT, v_c, v_m, inv_cnt,
      lp['gn_cart_w'].reshape(1, H), lp['gn_cart_b'].reshape(1, H),
      lp['gn_mat_w'].reshape(1, H), lp['gn_mat_b'].reshape(1, H),
      lp['mix_w'], lp['mix_b'].reshape(1, H))


def _readout_body(xc_r, xm_r, oh_r, ic_r, w1_r, b1_r, w2_r, b2_r, out_r):
    xf = (xc_r[...] + xm_r[...]) * 0.5
    feats = _DOT(oh_r[...], xf) * ic_r[...]
    h1 = _DOTBF(feats, w1_r[...]) + b1_r[...]
    h1 = h1 * jax.nn.sigmoid(h1)
    out_r[...] = _DOTBF(h1, w2_r[...]) + b2_r[...]


def _tc_readout(x_cart, x_mat, oh, inv_cnt, w1, b1, w2, b2):
    return pl.pallas_call(
        _readout_body,
        out_shape=jax.ShapeDtypeStruct((NGRAPH, 1), jnp.float32),
    )(x_cart, x_mat, oh, inv_cnt, w1, b1.reshape(1, H), w2, b2.reshape(1, 1))


def kernel(x_atom, edge_index, edge_attr, batch, params):
    src = edge_index[0].astype(jnp.int32)
    dst = edge_index[1].astype(jnp.int32)
    batch_i = jnp.pad(batch.astype(jnp.int32), (0, NP - N),
                      constant_values=NGRAPH)

    oh = (batch_i[None, :] == jnp.arange(NGRAPH, dtype=jnp.int32)[:, None]
          ).astype(jnp.float32)
    ohT = oh.T
    cnt = jnp.sum(oh, axis=1, keepdims=True)
    inv_cnt = 1.0 / jnp.maximum(cnt, 1.0)

    atom_p = jnp.pad(x_atom.astype(jnp.int32), (0, NP - N),
                     constant_values=-1)
    oh119 = (atom_p[:, None]
             == jnp.arange(119, dtype=jnp.int32)[None, :]).astype(jnp.float32)
    node_f, node_cmb = _tc_emb(oh119, params['atom_emb'])

    ef = _tc_ef(edge_attr, params['rbf_w1'], params['rbf_b1'],
                params['rbf_w2'], params['rbf_b2'])

    zeros_nh = jnp.zeros((NP, H), jnp.float32)

    x_cart = node_f
    x_mat = node_f
    x_cmb = node_cmb
    as_i32 = lambda xb: lax.bitcast_convert_type(
        xb.reshape(NP, H, 2), jnp.int32)
    for lp in params['layers']:
        gd, gs = _sc_gather2(as_i32(x_cmb), dst, src)
        mc, mm = _tc_edge(gd, gs, ef, lp)
        agg = _sc_scatter2(mc, mm, dst, zeros_nh)
        x_cart, x_mat, x_cmb = _tc_node(
            x_cart, x_mat, agg, oh, ohT, inv_cnt, lp)

    return _tc_readout(x_cart, x_mat, oh, inv_cnt,
                       params['fc1_w'], params['fc1_b'],
                       params['fc2_w'], params['fc2_b'])
